# baseline (speedup 1.0000x reference)
import functools

import jax
import jax.numpy as jnp
from jax import lax
from jax.experimental import pallas as pl
from jax.experimental.pallas import tpu as pltpu
from jax.experimental.pallas import tpu_sc as plsc

_THIRD = 1.0 / 3.0
_NC = 2
_NS = 16
_NW = _NC * _NS
_R = 240
_NGRP = _R // 16
_CPB = 8400 // _R
_BPW = 2
_NCHUNK = _CPB * _BPW

_TRIPLES = [(s, s + 10, s + 20) for s in list(range(10)) + list(range(30, 40))]


def _sc_stats_kernel(x_hbm, out_hbm, buf0, buf1, acc, sem0, sem1):
    cid = lax.axis_index("c")
    sid = lax.axis_index("s")
    wid = sid * _NC + cid
    b0 = wid * _BPW

    zeros16 = jnp.zeros((16,), jnp.float32)
    ninf16 = jnp.full((16,), -jnp.inf, jnp.float32)
    liota = lax.iota(jnp.int32, 16)

    for r in range(4):
        acc[r, pl.ds(0, 16)] = zeros16
        acc[r, pl.ds(16, 16)] = zeros16
    for r in range(4, 6):
        acc[r, pl.ds(0, 16)] = ninf16
        acc[r, pl.ds(16, 16)] = ninf16

    def chunk_src(k):
        batch = b0 + k // _CPB
        row0 = (k % _CPB) * _R
        return x_hbm.at[batch, pl.ds(row0, _R), :]

    def process(buf, k):
        del k
        for j, (s, t1, t2) in enumerate(_TRIPLES):
            cs = jnp.full((16,), s, jnp.int32)
            c1 = jnp.full((16,), t1, jnp.int32)
            c2 = jnp.full((16,), t2, jnp.int32)

            def grp(g, carry):
                p, q1, q2, e, m1, m2 = carry
                ridx = g * 16 + liota
                xs = plsc.load_gather(buf, [ridx, cs])
                x1 = plsc.load_gather(buf, [ridx, c1])
                x2 = plsc.load_gather(buf, [ridx, c2])
                ss = 1.0 / (1.0 + jnp.exp(-xs))
                s1 = 1.0 / (1.0 + jnp.exp(-x1))
                s2 = 1.0 / (1.0 + jnp.exp(-x2))
                om = 1.0 - ss
                v1 = s1 * om
                v2 = s2 * om
                ve = s1 * s2
                p = p + ss * ss * ss
                q1 = q1 + v1 * v1 * v1
                q2 = q2 + v2 * v2 * v2
                e = e + ve * ve * ve
                m1 = jnp.maximum(m1, x1)
                m2 = jnp.maximum(m2, x2)
                return p, q1, q2, e, m1, m2

            p, q1, q2, e, m1, m2 = lax.fori_loop(
                0, _NGRP, grp,
                (zeros16, zeros16, zeros16, zeros16, ninf16, ninf16))

            half = 16 * (j // 16)
            jj = j % 16
            for r, v in ((0, jnp.sum(p)), (1, jnp.sum(q1)),
                         (2, jnp.sum(q2)), (3, jnp.sum(e))):
                cur = acc[r, pl.ds(half, 16)]
                acc[r, pl.ds(half, 16)] = cur + jnp.where(liota == jj, v, 0.0)
            for r, v in ((4, jnp.max(m1)), (5, jnp.max(m2))):
                cur = acc[r, pl.ds(half, 16)]
                acc[r, pl.ds(half, 16)] = jnp.maximum(
                    cur, jnp.where(liota == jj, v, -jnp.inf))

    pltpu.async_copy(chunk_src(0), buf0, sem0).wait()

    def two_chunks(i, _):
        k0 = i * 2
        cp1 = pltpu.async_copy(chunk_src(k0 + 1), buf1, sem1)
        process(buf0, k0)
        cp1.wait()
        nxt = jnp.minimum(k0 + 2, _NCHUNK - 1)
        cp0 = pltpu.async_copy(chunk_src(nxt), buf0, sem0)
        process(buf1, k0 + 1)
        cp0.wait()
        return 0

    lax.fori_loop(0, _NCHUNK // 2, two_chunks, 0)

    pltpu.sync_copy(acc, out_hbm.at[wid])


def _combine_kernel(y_ref, out_ref, *, n_rows):
    y = y_ref[...]
    sums = jnp.sum(y[:, 0:4, :], axis=0)
    maxs = jnp.max(y[:, 4:6, :], axis=0)
    inv_n = 1.0 / n_rows
    roots = (sums * inv_n) ** _THIRD
    p3r = roots[0:1, :]
    qr = roots[1:2, :] + roots[2:3, :]
    er = roots[3:4, :]
    m1 = 1.0 / (1.0 + jnp.exp(-maxs[0:1, :]))
    m2 = 1.0 / (1.0 + jnp.exp(-maxs[1:2, :]))
    m = jnp.maximum(m1, m2)
    lane = lax.broadcasted_iota(jnp.int32, (1, 32), 1)
    valid = lane < 20
    picked = jnp.where(valid, 0.1 * ((1.0 - m) * p3r + er) + 0.05 * qr, 0.0)
    out_ref[...] = jnp.sum(picked, axis=1, keepdims=True)[0:1, 0:1]


def kernel(pred_scores, target_scores):
    del target_scores
    b, a, c = pred_scores.shape
    n_rows = b * a

    mesh = plsc.VectorSubcoreMesh(core_axis_name="c", subcore_axis_name="s")
    stats = pl.kernel(
        _sc_stats_kernel,
        mesh=mesh,
        compiler_params=pltpu.CompilerParams(needs_layout_passes=False),
        out_type=jax.ShapeDtypeStruct((_NW, 8, 32), jnp.float32),
        scratch_types=[
            pltpu.VMEM((_R, c), jnp.float32),
            pltpu.VMEM((_R, c), jnp.float32),
            pltpu.VMEM((8, 32), jnp.float32),
            pltpu.SemaphoreType.DMA,
            pltpu.SemaphoreType.DMA,
        ],
    )(pred_scores)

    out = pl.pallas_call(
        functools.partial(_combine_kernel, n_rows=n_rows),
        out_shape=jax.ShapeDtypeStruct((1, 1), jnp.float32),
    )(stats)
    return out.reshape(())
